# Initial kernel scaffold; baseline (speedup 1.0000x reference)
import jax
import jax.numpy as jnp
from jax.experimental import pallas as pl


def kernel(indices, table):
    raise NotImplementedError("write your pallas kernel here")


# speedup vs baseline: 1.8303x; 1.8303x over previous
import functools

import jax
import jax.numpy as jnp
from jax import lax
from jax.experimental import pallas as pl
from jax.experimental.pallas import tpu as pltpu
from jax.experimental.pallas import tpu_sc as plsc

NC = 2
NS = 16
NW = NC * NS

GROUP = 128
CHUNK = 4
ROWS = GROUP * CHUNK


@functools.partial(jax.jit, static_argnums=(2, 3))
def _sc_gather(idx, table, bpw, d):
    n_grp = bpw // GROUP
    n_chunk = n_grp // CHUNK
    mesh = plsc.VectorSubcoreMesh(core_axis_name="c", subcore_axis_name="s")

    @functools.partial(
        pl.kernel,
        out_type=jax.ShapeDtypeStruct((NW * bpw, d), jnp.float32),
        mesh=mesh,
        scratch_types=[
            pltpu.VMEM((n_grp, GROUP), jnp.int32),
            pltpu.VMEM((ROWS, d), jnp.float32),
            pltpu.SemaphoreType.DMA,
        ],
        compiler_params=pltpu.CompilerParams(use_tc_tiling_on_sc=False),
    )
    def k(idx_hbm, table_hbm, out_hbm, idx_v, rows_v, gsem):
        wid = lax.axis_index("s") * NC + lax.axis_index("c")
        base = wid * bpw
        pltpu.sync_copy(idx_hbm.at[wid], idx_v)

        @pl.loop(0, n_chunk)
        def _(cc):
            for j in range(CHUNK):
                pltpu.async_copy(
                    table_hbm.at[idx_v.at[cc * CHUNK + j]],
                    rows_v.at[pl.ds(j * GROUP, GROUP)],
                    gsem,
                )
            pltpu.make_async_copy(
                out_hbm.at[pl.ds(base, ROWS)], rows_v, gsem
            ).wait()
            pltpu.sync_copy(rows_v, out_hbm.at[pl.ds(base + cc * ROWS, ROWS)])

    return k(idx, table)


def kernel(indices, table):
    batch, hist = indices.shape
    _, d = table.shape
    b_total = batch * hist
    bpw = b_total // NW
    idx = indices.reshape(NW, bpw // GROUP, GROUP)
    out = _sc_gather(idx, table, bpw, d)
    return out.reshape(batch, hist, d)


# speedup vs baseline: 1.8735x; 1.0236x over previous
import functools

import jax
import jax.numpy as jnp
from jax import lax
from jax.experimental import pallas as pl
from jax.experimental.pallas import tpu as pltpu
from jax.experimental.pallas import tpu_sc as plsc

NC = 2
NS = 16
NW = NC * NS

GROUP = 128
CHUNK = 4
ROWS = GROUP * CHUNK


@functools.partial(jax.jit, static_argnums=(2, 3))
def _sc_gather(idx, table, bpw, d):
    n_grp = bpw // GROUP
    n_chunk = n_grp // CHUNK
    mesh = plsc.VectorSubcoreMesh(core_axis_name="c", subcore_axis_name="s")

    @functools.partial(
        pl.kernel,
        out_type=jax.ShapeDtypeStruct((NW * bpw, d), jnp.float32),
        mesh=mesh,
        scratch_types=[
            pltpu.VMEM((n_grp, GROUP), jnp.int32),
            pltpu.VMEM((2, ROWS, d), jnp.float32),
            pltpu.SemaphoreType.DMA,
            pltpu.SemaphoreType.DMA,
            pltpu.SemaphoreType.DMA,
            pltpu.SemaphoreType.DMA,
        ],
        compiler_params=pltpu.CompilerParams(use_tc_tiling_on_sc=False),
    )
    def k(idx_hbm, table_hbm, out_hbm, idx_v, rows_v, gsem0, gsem1, wsem0, wsem1):
        gsem = (gsem0, gsem1)
        wsem = (wsem0, wsem1)
        wid = lax.axis_index("s") * NC + lax.axis_index("c")
        base = wid * bpw
        pltpu.sync_copy(idx_hbm.at[wid], idx_v)

        def fire_gathers(cc, b):
            for j in range(CHUNK):
                pltpu.async_copy(
                    table_hbm.at[idx_v.at[cc * CHUNK + j]],
                    rows_v.at[b, pl.ds(j * GROUP, GROUP)],
                    gsem[b],
                )

        def wait_gathers(b):
            pltpu.make_async_copy(
                out_hbm.at[pl.ds(base, ROWS)], rows_v.at[b], gsem[b]
            ).wait()

        def fire_write(cc, b):
            pltpu.async_copy(
                rows_v.at[b], out_hbm.at[pl.ds(base + cc * ROWS, ROWS)], wsem[b]
            )

        def wait_write(b):
            pltpu.make_async_copy(
                rows_v.at[b], out_hbm.at[pl.ds(base, ROWS)], wsem[b]
            ).wait()

        fire_gathers(0, 0)

        @pl.loop(0, n_chunk, step=2)
        def _(c):
            @pl.when(c > 0)
            def _():
                wait_write(1)

            fire_gathers(c + 1, 1)
            wait_gathers(0)
            fire_write(c, 0)

            @pl.when(c + 2 < n_chunk)
            def _():
                wait_write(0)
                fire_gathers(c + 2, 0)

            wait_gathers(1)
            fire_write(c + 1, 1)

        wait_write(0)
        wait_write(1)

    return k(idx, table)


def kernel(indices, table):
    batch, hist = indices.shape
    _, d = table.shape
    b_total = batch * hist
    bpw = b_total // NW
    idx = indices.reshape(NW, bpw // GROUP, GROUP)
    out = _sc_gather(idx, table, bpw, d)
    return out.reshape(batch, hist, d)
